# Initial kernel scaffold; baseline (speedup 1.0000x reference)
import jax
import jax.numpy as jnp
from jax.experimental import pallas as pl


def kernel(text, word2state, preterminal_emb, terminal_emb, W1, b1, W2, b2):
    raise NotImplementedError("write your pallas kernel here")


# speedup vs baseline: 3.2088x; 3.2088x over previous
import functools

import jax
import jax.numpy as jnp
from jax import lax
from jax.experimental import pallas as pl
from jax.experimental.pallas import tpu as pltpu
from jax.experimental.pallas import tpu_sc as plsc

C = 1024
H = 128
SPW = 16
LANES = 16
NCHUNK = H // LANES

_info = plsc.get_sparse_core_info()
NC, NS = _info.num_cores, _info.num_subcores
NW = NC * NS


def _r_table_body(p_ref, w1_ref, b1_ref, w2_ref, b2_ref, r_ref):
    p = p_ref[...]
    h = p + jnp.maximum(
        jnp.dot(p, w1_ref[...], preferred_element_type=jnp.float32) + b1_ref[...],
        0.0,
    )
    r_ref[...] = h + jnp.maximum(
        jnp.dot(h, w2_ref[...], preferred_element_type=jnp.float32) + b2_ref[...],
        0.0,
    )


def _compute_r_table(pret, w1, b1, w2, b2):
    return pl.pallas_call(
        _r_table_body,
        out_shape=jax.ShapeDtypeStruct((C, H), jnp.float32),
    )(pret, w1, b1.reshape(1, H), w2, b2.reshape(1, H))


def _log_splat(s):
    sv = jnp.broadcast_to(s, (LANES,))
    bits = plsc.bitcast(sv, jnp.int32)
    ex = jnp.right_shift(bits, 23) - 127
    mant = plsc.bitcast(
        jnp.bitwise_or(jnp.bitwise_and(bits, 0x7FFFFF), 0x3F800000), jnp.float32
    )
    big = mant > 1.4142135
    mant = jnp.where(big, mant * 0.5, mant)
    ex = ex + jnp.where(big, 1, 0)
    t = mant - 1.0
    p = 1.0 / 8.0 - t * (1.0 / 9.0)
    p = 1.0 / 7.0 - t * p
    p = 1.0 / 6.0 - t * p
    p = 1.0 / 5.0 - t * p
    p = 1.0 / 4.0 - t * p
    p = 1.0 / 3.0 - t * p
    p = 1.0 / 2.0 - t * p
    p = t * (1.0 - t * p)
    return ex.astype(jnp.float32) * 0.69314718 + p


def _make_sc_kernel(n_tokens):
    tpw = n_tokens // NW
    grp = 8
    ngrp = tpw // grp
    mesh = plsc.VectorSubcoreMesh(core_axis_name="c", subcore_axis_name="s")

    @functools.partial(
        pl.kernel,
        mesh=mesh,
        out_type=jax.ShapeDtypeStruct((n_tokens, SPW), jnp.float32),
        scratch_types=[
            pltpu.VMEM((tpw,), jnp.int32),
            pltpu.VMEM((tpw, SPW), jnp.int32),
            pltpu.VMEM((tpw * SPW,), jnp.int32),
            pltpu.VMEM((tpw, H), jnp.float32),
            pltpu.VMEM((grp * SPW, H), jnp.float32),
            pltpu.VMEM((tpw, SPW), jnp.float32),
            pltpu.SemaphoreType.DMA,
        ],
        compiler_params=pltpu.CompilerParams(
            needs_layout_passes=False, use_tc_tiling_on_sc=False
        ),
    )
    def sc_kernel(text_hbm, w2s_hbm, r_hbm, emb_hbm, out_hbm,
                  idx_v, st2_v, stf_v, obs_v, rr_v, res_v, sem):
        wid = lax.axis_index("s") * NC + lax.axis_index("c")
        base = wid * tpw
        pltpu.sync_copy(text_hbm.at[pl.ds(base, tpw)], idx_v)
        pltpu.async_copy(w2s_hbm.at[idx_v], st2_v, sem).wait()
        pltpu.async_copy(emb_hbm.at[idx_v], obs_v, sem).wait()

        def flatten_body(t, carry):
            stf_v[pl.ds(t * SPW, SPW)] = st2_v[t]
            return carry

        lax.fori_loop(0, tpw, flatten_body, 0)

        lanes = lax.iota(jnp.int32, LANES)

        def group_body(g, carry):
            pltpu.async_copy(
                r_hbm.at[stf_v.at[pl.ds(g * (grp * SPW), grp * SPW)]], rr_v, sem
            ).wait()

            def token_body(j, carry2):
                t = g * grp + j
                o = [obs_v[t, pl.ds(c * LANES, LANES)] for c in range(NCHUNK)]
                logits = jnp.zeros((LANES,), jnp.float32)
                for k in range(SPW):
                    row = j * SPW + k
                    acc = o[0] * rr_v[row, pl.ds(0, LANES)]
                    for c in range(1, NCHUNK):
                        acc = acc + o[c] * rr_v[row, pl.ds(c * LANES, LANES)]
                    logits = jnp.where(lanes == k, jnp.sum(acc), logits)
                m = jnp.max(logits)
                x = logits - m
                s = jnp.sum(jnp.exp(x))
                res_v[t] = x - _log_splat(s)
                return carry2

            lax.fori_loop(0, grp, token_body, 0)
            return carry

        lax.fori_loop(0, ngrp, group_body, 0)
        pltpu.sync_copy(res_v, out_hbm.at[pl.ds(base, tpw)])

    return sc_kernel


def kernel(text, word2state, preterminal_emb, terminal_emb, W1, b1, W2, b2):
    b, t = text.shape
    n = b * t
    r_table = _compute_r_table(preterminal_emb, W1, b1, W2, b2)
    flat_text = text.reshape(n).astype(jnp.int32)
    out = _make_sc_kernel(n)(flat_text, word2state, r_table, terminal_emb)
    return out.reshape(b, t, SPW)


# speedup vs baseline: 3.4204x; 1.0659x over previous
import functools

import jax
import jax.numpy as jnp
from jax import lax
from jax.experimental import pallas as pl
from jax.experimental.pallas import tpu as pltpu
from jax.experimental.pallas import tpu_sc as plsc

C = 1024
H = 128
SPW = 16
LANES = 16
NCHUNK = H // LANES

_info = plsc.get_sparse_core_info()
NC, NS = _info.num_cores, _info.num_subcores
NW = NC * NS


def _r_table_body(p_ref, w1_ref, b1_ref, w2_ref, b2_ref, r_ref):
    p = p_ref[...]
    h = p + jnp.maximum(
        jnp.dot(p, w1_ref[...], preferred_element_type=jnp.float32) + b1_ref[...],
        0.0,
    )
    r_ref[...] = h + jnp.maximum(
        jnp.dot(h, w2_ref[...], preferred_element_type=jnp.float32) + b2_ref[...],
        0.0,
    )


def _compute_r_table(pret, w1, b1, w2, b2):
    return pl.pallas_call(
        _r_table_body,
        out_shape=jax.ShapeDtypeStruct((C, H), jnp.float32),
    )(pret, w1, b1.reshape(1, H), w2, b2.reshape(1, H))


def _log_vec(sv):
    bits = plsc.bitcast(sv, jnp.int32)
    ex = jnp.right_shift(bits, 23) - 127
    mant = plsc.bitcast(
        jnp.bitwise_or(jnp.bitwise_and(bits, 0x7FFFFF), 0x3F800000), jnp.float32
    )
    big = mant > 1.4142135
    mant = jnp.where(big, mant * 0.5, mant)
    ex = ex + jnp.where(big, 1, 0)
    t = mant - 1.0
    p = 1.0 / 8.0 - t * (1.0 / 9.0)
    p = 1.0 / 7.0 - t * p
    p = 1.0 / 6.0 - t * p
    p = 1.0 / 5.0 - t * p
    p = 1.0 / 4.0 - t * p
    p = 1.0 / 3.0 - t * p
    p = 1.0 / 2.0 - t * p
    p = t * (1.0 - t * p)
    return ex.astype(jnp.float32) * 0.69314718 + p


def _tree_sum(vs):
    while len(vs) > 1:
        vs = [vs[i] + vs[i + 1] for i in range(0, len(vs) - 1, 2)] + (
            [vs[-1]] if len(vs) % 2 else []
        )
    return vs[0]


def _tree_max(vs):
    while len(vs) > 1:
        vs = [jnp.maximum(vs[i], vs[i + 1]) for i in range(0, len(vs) - 1, 2)] + (
            [vs[-1]] if len(vs) % 2 else []
        )
    return vs[0]


def _make_sc_kernel(n_tokens):
    tpw = n_tokens // NW
    blk = 16
    nblk = tpw // blk
    mesh = plsc.VectorSubcoreMesh(core_axis_name="c", subcore_axis_name="s")

    @functools.partial(
        pl.kernel,
        mesh=mesh,
        out_type=jax.ShapeDtypeStruct((n_tokens, SPW), jnp.float32),
        scratch_types=[
            pltpu.VMEM((tpw,), jnp.int32),
            pltpu.VMEM((tpw, SPW), jnp.int32),
            pltpu.VMEM((tpw * SPW,), jnp.int32),
            pltpu.VMEM((tpw, H), jnp.float32),
            pltpu.VMEM((blk * SPW, H), jnp.float32),
            pltpu.VMEM((blk * SPW, H), jnp.float32),
            pltpu.VMEM((SPW, LANES), jnp.float32),
            pltpu.VMEM((blk, SPW), jnp.float32),
            pltpu.VMEM((tpw, SPW), jnp.float32),
            pltpu.SemaphoreType.DMA,
            pltpu.SemaphoreType.DMA,
        ],
        compiler_params=pltpu.CompilerParams(
            needs_layout_passes=False, use_tc_tiling_on_sc=False
        ),
    )
    def sc_kernel(text_hbm, w2s_hbm, r_hbm, emb_hbm, out_hbm,
                  idx_v, st2_v, stf_v, obs_v, rr0_v, rr1_v, tr_v, lt_v, res_v,
                  sem0, sem1):
        wid = lax.axis_index("s") * NC + lax.axis_index("c")
        base = wid * tpw
        pltpu.sync_copy(text_hbm.at[pl.ds(base, tpw)], idx_v)
        pltpu.async_copy(w2s_hbm.at[idx_v], st2_v, sem0).wait()
        pltpu.async_copy(emb_hbm.at[idx_v], obs_v, sem0).wait()

        def flatten_body(t, carry):
            stf_v[pl.ds(t * SPW, SPW)] = st2_v[t]
            return carry

        lax.fori_loop(0, tpw, flatten_body, 0)

        lanes = lax.iota(jnp.int32, LANES)
        half = blk * SPW // 2

        def start_blk(b, rr, sem):
            off = b * blk * SPW
            pltpu.async_copy(
                r_hbm.at[stf_v.at[pl.ds(off, half)]], rr.at[pl.ds(0, half)], sem
            )
            pltpu.async_copy(
                r_hbm.at[stf_v.at[pl.ds(off + half, half)]],
                rr.at[pl.ds(half, half)], sem,
            )

        def wait_blk(b, rr, sem):
            off = b * blk * SPW
            pltpu.make_async_copy(
                r_hbm.at[stf_v.at[pl.ds(off, half)]], rr.at[pl.ds(0, half)], sem
            ).wait()
            pltpu.make_async_copy(
                r_hbm.at[stf_v.at[pl.ds(off + half, half)]],
                rr.at[pl.ds(half, half)], sem,
            ).wait()

        def compute_blk(b, rr):
            def token_body(j, carry):
                t = b * blk + j
                o = [obs_v[t, pl.ds(c * LANES, LANES)] for c in range(NCHUNK)]
                for k in range(SPW):
                    tr_v[k] = _tree_sum(
                        [o[c] * rr[j * SPW + k, pl.ds(c * LANES, LANES)]
                         for c in range(NCHUNK)]
                    )
                cols = [
                    plsc.load_gather(tr_v, [lanes, jnp.full((LANES,), c, jnp.int32)])
                    for c in range(LANES)
                ]
                lt_v[j] = _tree_sum(cols)
                return carry

            lax.fori_loop(0, blk, token_body, 0)

            vks = [
                plsc.load_gather(lt_v, [lanes, jnp.full((LANES,), k, jnp.int32)])
                for k in range(SPW)
            ]
            m = _tree_max(vks)
            xs = [vk - m for vk in vks]
            ls = _log_vec(_tree_sum([jnp.exp(x) for x in xs]))
            rows = b * blk + lanes
            for k in range(SPW):
                plsc.store_scatter(
                    res_v, [rows, jnp.full((LANES,), k, jnp.int32)], xs[k] - ls
                )

        start_blk(0, rr0_v, sem0)
        start_blk(1, rr1_v, sem1)

        def pair_body(p, carry):
            b0 = 2 * p
            b1 = 2 * p + 1
            wait_blk(b0, rr0_v, sem0)
            compute_blk(b0, rr0_v)

            @pl.when(b0 + 2 < nblk)
            def _():
                start_blk(b0 + 2, rr0_v, sem0)

            wait_blk(b1, rr1_v, sem1)
            compute_blk(b1, rr1_v)

            @pl.when(b1 + 2 < nblk)
            def _():
                start_blk(b1 + 2, rr1_v, sem1)

            return carry

        lax.fori_loop(0, nblk // 2, pair_body, 0)
        pltpu.sync_copy(res_v, out_hbm.at[pl.ds(base, tpw)])

    return sc_kernel


def kernel(text, word2state, preterminal_emb, terminal_emb, W1, b1, W2, b2):
    b, t = text.shape
    n = b * t
    r_table = _compute_r_table(preterminal_emb, W1, b1, W2, b2)
    flat_text = text.reshape(n).astype(jnp.int32)
    out = _make_sc_kernel(n)(flat_text, word2state, r_table, terminal_emb)
    return out.reshape(b, t, SPW)
